# Initial kernel scaffold; baseline (speedup 1.0000x reference)
import functools

import jax
import jax.numpy as jnp
from jax import lax
from jax.experimental import pallas as pl
from jax.experimental.pallas import tpu as pltpu
from jax.experimental.pallas import tpu_sc as plsc

N = 10000
E = 320000
D_IN = 128
D_HID = 64
N_CLASS = 40
NEG_SLOPE = 0.2

NC = 2
NS = 16
NW = NC * NS
L = 16

EPT = E // NW
K = 128
NB = EPT // K
TAIL = EPT - NB * K
RPT = N // NS


def _sc_aggregate(de):
    mesh = plsc.VectorSubcoreMesh(core_axis_name="c", subcore_axis_name="s")
    nch = de // L

    @functools.partial(
        pl.kernel,
        out_type=[
            jax.ShapeDtypeStruct((NC, N, de), jnp.float32),
            jax.ShapeDtypeStruct((NW, N), jnp.float32),
        ],
        mesh=mesh,
        scratch_types=[
            pltpu.VMEM((N, 2), jnp.float32),
            pltpu.VMEM((16,), jnp.float32),
            pltpu.VMEM((N,), jnp.float32),
            pltpu.VMEM((K,), jnp.int32),
            pltpu.VMEM((K,), jnp.int32),
            pltpu.VMEM((K,), jnp.float32),
            pltpu.VMEM((K, de), jnp.float32),
            pltpu.VMEM((TAIL,), jnp.int32),
            pltpu.VMEM((TAIL,), jnp.int32),
            pltpu.VMEM((TAIL, de), jnp.float32),
            pltpu.VMEM_SHARED((N, de), jnp.float32),
        ],
    )
    def agg(h_hbm, scal_hbm, src_hbm, dst_hbm, m_hbm, out_hbm, den_hbm,
            scal_v, m_v, den_v, src_v, dst_v, w_v, rows_v,
            src_t, dst_t, rows_t, acc_sh):
        c = lax.axis_index("c")
        s = lax.axis_index("s")
        wid = c * NS + s

        pltpu.sync_copy(scal_hbm, scal_v)
        pltpu.sync_copy(m_hbm, m_v)
        mvec = m_v[...]

        zf = jnp.zeros((L,), jnp.float32)
        zi = jnp.zeros((L,), jnp.int32)
        oi = jnp.ones((L,), jnp.int32)

        @pl.loop(0, N // L)
        def _(i):
            den_v[pl.ds(i * L, L)] = zf

        @pl.loop(0, K)
        def _(k):
            for ch in range(nch):
                rows_v[k, pl.ds(ch * L, L)] = zf

        nfull = RPT // K

        @pl.loop(0, nfull)
        def _(i):
            pltpu.sync_copy(rows_v, acc_sh.at[pl.ds(s * RPT + i * K, K)])

        rem = RPT - nfull * K
        pltpu.sync_copy(rows_v.at[pl.ds(0, rem)],
                        acc_sh.at[pl.ds(s * RPT + nfull * K, rem)])
        plsc.subcore_barrier()

        def weights(kb, srcr, dstr):
            for j in range(kb // L):
                si = srcr[pl.ds(j * L, L)]
                di = dstr[pl.ds(j * L, L)]
                es = plsc.load_gather(scal_v, [si, zi])
                ed = plsc.load_gather(scal_v, [di, oi])
                e = es + ed
                e = jnp.where(e > 0.0, e, NEG_SLOPE * e)
                w = jnp.exp(e - mvec)
                w_v[pl.ds(j * L, L)] = w
                plsc.addupdate_scatter(den_v, [di], w)

        def do_block(base, kb, srcr, dstr, rowsr):
            pltpu.sync_copy(src_hbm.at[pl.ds(base, kb)], srcr)
            pltpu.sync_copy(dst_hbm.at[pl.ds(base, kb)], dstr)
            pltpu.sync_copy(h_hbm.at[srcr], rowsr)
            weights(kb, srcr, dstr)

            @pl.loop(0, kb)
            def _(k):
                wk = w_v[k]
                for ch in range(nch):
                    sl = pl.ds(ch * L, L)
                    rowsr[k, sl] = rowsr[k, sl] * wk

            pltpu.sync_copy(rowsr, acc_sh.at[dstr], add=True)

        ebase = wid * EPT

        @pl.loop(0, NB)
        def _(b):
            do_block(ebase + b * K, K, src_v, dst_v, rows_v)

        do_block(ebase + NB * K, TAIL, src_t, dst_t, rows_t)

        pltpu.sync_copy(den_v, den_hbm.at[wid])
        plsc.subcore_barrier()
        pltpu.sync_copy(acc_sh.at[pl.ds(s * RPT, RPT)],
                        out_hbm.at[c, pl.ds(s * RPT, RPT)])

    return agg


def _tc_front(x, w1, a1):
    def body(x_ref, w_ref, a_ref, h_ref, scal_ref, m_ref):
        h = jnp.dot(x_ref[...], w_ref[...],
                    preferred_element_type=jnp.float32,
                    precision=lax.Precision.HIGHEST)
        h_ref[...] = h
        sc = jnp.dot(h, a_ref[...], preferred_element_type=jnp.float32,
                     precision=lax.Precision.HIGHEST)
        scal_ref[...] = sc
        m = jnp.maximum(jnp.max(sc[:, 0]) + jnp.max(sc[:, 1]), 0.0)
        m_ref[...] = jnp.full((1, L), m, jnp.float32)

    return pl.pallas_call(
        body,
        out_shape=[
            jax.ShapeDtypeStruct((N, D_HID), jnp.float32),
            jax.ShapeDtypeStruct((N, 2), jnp.float32),
            jax.ShapeDtypeStruct((1, L), jnp.float32),
        ],
    )(x, w1, a1)


def _tc_mid(out1, den1, w2, a2, de2):
    def body(o_ref, d_ref, w_ref, a_ref, h_ref, scal_ref, m_ref):
        t = o_ref[0] + o_ref[1]
        den = jnp.sum(d_ref[...], axis=0)
        h1 = t / jnp.maximum(den, 1e-30)[:, None]
        h1 = jnp.where(h1 > 0.0, h1, jnp.exp(h1) - 1.0)
        h2 = jnp.dot(h1, w_ref[...], preferred_element_type=jnp.float32,
                     precision=lax.Precision.HIGHEST)
        h_ref[...] = jnp.concatenate(
            [h2, jnp.zeros((N, de2 - N_CLASS), jnp.float32)], axis=1)
        sc = jnp.dot(h2, a_ref[...], preferred_element_type=jnp.float32,
                     precision=lax.Precision.HIGHEST)
        scal_ref[...] = sc
        m = jnp.maximum(jnp.max(sc[:, 0]) + jnp.max(sc[:, 1]), 0.0)
        m_ref[...] = jnp.full((1, L), m, jnp.float32)

    return pl.pallas_call(
        body,
        out_shape=[
            jax.ShapeDtypeStruct((N, de2), jnp.float32),
            jax.ShapeDtypeStruct((N, 2), jnp.float32),
            jax.ShapeDtypeStruct((1, L), jnp.float32),
        ],
    )(out1, den1, w2, a2)


def _tc_back(out2, den2):
    def body(o_ref, d_ref, out_ref):
        t = (o_ref[0] + o_ref[1])[:, :N_CLASS]
        den = jnp.sum(d_ref[...], axis=0)
        h2 = t / jnp.maximum(den, 1e-30)[:, None]
        mx = jnp.max(h2, axis=1, keepdims=True)
        z = h2 - mx
        lse = jnp.log(jnp.sum(jnp.exp(z), axis=1, keepdims=True))
        out_ref[...] = z - lse

    return pl.pallas_call(
        body,
        out_shape=jax.ShapeDtypeStruct((N, N_CLASS), jnp.float32),
    )(out2, den2)


def kernel(X, A, W1, a1_src, a1_dst, W2, a2_src, a2_dst):
    src = A[0]
    dst = A[1]
    a1 = jnp.stack([a1_src, a1_dst], axis=1)
    a2 = jnp.stack([a2_src, a2_dst], axis=1)

    h1, scal1, m1 = _tc_front(X, W1, a1)
    out1, den1 = _sc_aggregate(D_HID)(
        h1, scal1, src, dst, m1.reshape(L))

    de2 = 48
    h2, scal2, m2 = _tc_mid(out1, den1, W2, a2, de2)
    out2, den2 = _sc_aggregate(de2)(
        h2, scal2, src, dst, m2.reshape(L))

    return _tc_back(out2, den2)


# speedup vs baseline: 28.8282x; 28.8282x over previous
import dataclasses
import functools

import jax
import jax.numpy as jnp
from jax import lax
from jax.experimental import pallas as pl
from jax.experimental.pallas import tpu as pltpu
from jax.experimental.pallas import tpu_sc as plsc

N = 10000
E = 320000
D_IN = 128
D_HID = 64
N_CLASS = 40
NEG_SLOPE = 0.2

NC = 2
NS = 16
NW = NC * NS
L = 16

EPT = E // NW
K = 128
NB = EPT // K
TAIL = EPT - NB * K
WB = 624
WB_REM = N - NS * WB


def _sc_aggregate(de):
    mesh = plsc.VectorSubcoreMesh(core_axis_name="c", subcore_axis_name="s")
    nch = de // L

    cp = pltpu.CompilerParams()
    for fld, val in (("needs_layout_passes", False),
                     ("use_tc_tiling_on_sc", False)):
        if fld in pltpu.CompilerParams.__dataclass_fields__:
            cp = dataclasses.replace(cp, **{fld: val})

    @functools.partial(
        pl.kernel,
        compiler_params=cp,
        out_type=[
            jax.ShapeDtypeStruct((NC, N, de), jnp.float32),
            jax.ShapeDtypeStruct((NW, N), jnp.float32),
        ],
        mesh=mesh,
        scratch_types=[
            pltpu.VMEM((2 * N,), jnp.float32),
            pltpu.VMEM((16,), jnp.float32),
            pltpu.VMEM((N,), jnp.float32),
            pltpu.VMEM((K,), jnp.int32),
            pltpu.VMEM((K,), jnp.int32),
            pltpu.VMEM((K,), jnp.float32),
            pltpu.VMEM((K, de), jnp.float32),
            pltpu.VMEM((TAIL,), jnp.int32),
            pltpu.VMEM((TAIL,), jnp.int32),
            pltpu.VMEM((TAIL, de), jnp.float32),
            pltpu.VMEM_SHARED((N, de), jnp.float32),
        ],
    )
    def agg(h_hbm, scal_hbm, src_hbm, dst_hbm, m_hbm, out_hbm, den_hbm,
            scal_v, m_v, den_v, src_v, dst_v, w_v, rows_v,
            src_t, dst_t, rows_t, acc_sh):
        c = lax.axis_index("c")
        s = lax.axis_index("s")
        wid = c * NS + s

        pltpu.sync_copy(scal_hbm, scal_v)
        pltpu.sync_copy(m_hbm, m_v)
        mvec = m_v[...]

        zf = jnp.zeros((L,), jnp.float32)
        oi = jnp.ones((L,), jnp.int32)

        @pl.loop(0, N // L)
        def _(i):
            den_v[pl.ds(i * L, L)] = zf

        @pl.loop(0, K)
        def _(k):
            for ch in range(nch):
                rows_v[k, pl.ds(ch * L, L)] = zf

        nfull = WB // K

        @pl.loop(0, nfull)
        def _(i):
            pltpu.sync_copy(rows_v, acc_sh.at[pl.ds(s * WB + i * K, K)])

        rem = WB - nfull * K
        pltpu.sync_copy(rows_v.at[pl.ds(0, rem)],
                        acc_sh.at[pl.ds(s * WB + nfull * K, rem)])

        @pl.when(s == 0)
        def _():
            pltpu.sync_copy(rows_v.at[pl.ds(0, WB_REM)],
                            acc_sh.at[pl.ds(NS * WB, WB_REM)])

        plsc.subcore_barrier()

        def weights(kb, srcr, dstr):
            for j in range(kb // L):
                si = srcr[pl.ds(j * L, L)]
                di = dstr[pl.ds(j * L, L)]
                es = plsc.load_gather(scal_v, [si * 2])
                ed = plsc.load_gather(scal_v, [di * 2 + oi])
                e = es + ed
                e = jnp.where(e > 0.0, e, NEG_SLOPE * e)
                w = jnp.exp(e - mvec)
                w_v[pl.ds(j * L, L)] = w
                plsc.addupdate_scatter(den_v, [di], w)

        def do_block(base, kb, srcr, dstr, rowsr):
            pltpu.sync_copy(src_hbm.at[pl.ds(base, kb)], srcr)
            pltpu.sync_copy(dst_hbm.at[pl.ds(base, kb)], dstr)
            pltpu.sync_copy(h_hbm.at[srcr], rowsr)
            weights(kb, srcr, dstr)

            @pl.loop(0, kb // L)
            def _(j):
                wv = w_v[pl.ds(j * L, L)]
                for t in range(L):
                    wk = wv[t]
                    for ch in range(nch):
                        sl = pl.ds(ch * L, L)
                        rowsr[j * L + t, sl] = rowsr[j * L + t, sl] * wk

            pltpu.sync_copy(rowsr, acc_sh.at[dstr], add=True)

        ebase = wid * EPT

        @pl.loop(0, NB)
        def _(b):
            do_block(ebase + b * K, K, src_v, dst_v, rows_v)

        do_block(ebase + NB * K, TAIL, src_t, dst_t, rows_t)

        pltpu.sync_copy(den_v, den_hbm.at[wid])
        plsc.subcore_barrier()
        pltpu.sync_copy(acc_sh.at[pl.ds(s * WB, WB)],
                        out_hbm.at[c, pl.ds(s * WB, WB)])

        @pl.when(s == 0)
        def _():
            pltpu.sync_copy(acc_sh.at[pl.ds(NS * WB, WB_REM)],
                            out_hbm.at[c, pl.ds(NS * WB, WB_REM)])

    return agg


def _tc_front(x, w1, a1):
    def body(x_ref, w_ref, a_ref, h_ref, scal_ref, m_ref):
        h = jnp.dot(x_ref[...], w_ref[...],
                    preferred_element_type=jnp.float32,
                    precision=lax.Precision.HIGHEST)
        h_ref[...] = h
        sc = jnp.dot(h, a_ref[...], preferred_element_type=jnp.float32,
                     precision=lax.Precision.HIGHEST)
        scal_ref[...] = sc
        m = jnp.maximum(jnp.max(sc[:, 0]) + jnp.max(sc[:, 1]), 0.0)
        m_ref[...] = jnp.full((1, L), m, jnp.float32)

    return pl.pallas_call(
        body,
        out_shape=[
            jax.ShapeDtypeStruct((N, D_HID), jnp.float32),
            jax.ShapeDtypeStruct((N, 2), jnp.float32),
            jax.ShapeDtypeStruct((1, L), jnp.float32),
        ],
    )(x, w1, a1)


def _tc_mid(out1, den1, w2, a2, de2):
    def body(o_ref, d_ref, w_ref, a_ref, h_ref, scal_ref, m_ref):
        t = o_ref[0] + o_ref[1]
        den = jnp.sum(d_ref[...], axis=0)
        h1 = t / jnp.maximum(den, 1e-30)[:, None]
        h1 = jnp.where(h1 > 0.0, h1, jnp.exp(h1) - 1.0)
        h2 = jnp.dot(h1, w_ref[...], preferred_element_type=jnp.float32,
                     precision=lax.Precision.HIGHEST)
        h_ref[...] = jnp.concatenate(
            [h2, jnp.zeros((N, de2 - N_CLASS), jnp.float32)], axis=1)
        sc = jnp.dot(h2, a_ref[...], preferred_element_type=jnp.float32,
                     precision=lax.Precision.HIGHEST)
        scal_ref[...] = sc
        m = jnp.maximum(jnp.max(sc[:, 0]) + jnp.max(sc[:, 1]), 0.0)
        m_ref[...] = jnp.full((1, L), m, jnp.float32)

    return pl.pallas_call(
        body,
        out_shape=[
            jax.ShapeDtypeStruct((N, de2), jnp.float32),
            jax.ShapeDtypeStruct((N, 2), jnp.float32),
            jax.ShapeDtypeStruct((1, L), jnp.float32),
        ],
    )(out1, den1, w2, a2)


def _tc_back(out2, den2):
    def body(o_ref, d_ref, out_ref):
        t = (o_ref[0] + o_ref[1])[:, :N_CLASS]
        den = jnp.sum(d_ref[...], axis=0)
        h2 = t / jnp.maximum(den, 1e-30)[:, None]
        mx = jnp.max(h2, axis=1, keepdims=True)
        z = h2 - mx
        lse = jnp.log(jnp.sum(jnp.exp(z), axis=1, keepdims=True))
        out_ref[...] = z - lse

    return pl.pallas_call(
        body,
        out_shape=jax.ShapeDtypeStruct((N, N_CLASS), jnp.float32),
    )(out2, den2)


def kernel(X, A, W1, a1_src, a1_dst, W2, a2_src, a2_dst):
    src = A[0]
    dst = A[1]
    a1 = jnp.stack([a1_src, a1_dst], axis=1)
    a2 = jnp.stack([a2_src, a2_dst], axis=1)

    h1, scal1, m1 = _tc_front(X, W1, a1)
    out1, den1 = _sc_aggregate(D_HID)(
        h1, scal1.reshape(2 * N), src, dst, m1.reshape(L))

    de2 = 48
    h2, scal2, m2 = _tc_mid(out1, den1, W2, a2, de2)
    out2, den2 = _sc_aggregate(de2)(
        h2, scal2.reshape(2 * N), src, dst, m2.reshape(L))

    return _tc_back(out2, den2)


# speedup vs baseline: 46.2536x; 1.6045x over previous
import dataclasses
import functools

import jax
import jax.numpy as jnp
from jax import lax
from jax.experimental import pallas as pl
from jax.experimental.pallas import tpu as pltpu
from jax.experimental.pallas import tpu_sc as plsc

N = 10000
E = 320000
D_IN = 128
D_HID = 64
N_CLASS = 40
NEG_SLOPE = 0.2

NC = 2
NS = 16
NW = NC * NS
L = 16

EPT = E // NW
K = 128
NB = EPT // K
TAIL = EPT - NB * K
WB = 624
WB_REM = N - NS * WB


def _sc_aggregate(de):
    mesh = plsc.VectorSubcoreMesh(core_axis_name="c", subcore_axis_name="s")
    nch = de // L

    cp = pltpu.CompilerParams()
    for fld, val in (("needs_layout_passes", False),
                     ("use_tc_tiling_on_sc", False)):
        if fld in pltpu.CompilerParams.__dataclass_fields__:
            cp = dataclasses.replace(cp, **{fld: val})

    @functools.partial(
        pl.kernel,
        compiler_params=cp,
        out_type=[
            jax.ShapeDtypeStruct((NC, N, de), jnp.float32),
            jax.ShapeDtypeStruct((NW, N), jnp.float32),
        ],
        mesh=mesh,
        scratch_types=[
            pltpu.VMEM((2 * N,), jnp.float32),
            pltpu.VMEM((16,), jnp.float32),
            pltpu.VMEM((N,), jnp.float32),
            pltpu.VMEM((EPT,), jnp.int32),
            pltpu.VMEM((K,), jnp.int32),
            pltpu.VMEM((K,), jnp.int32),
            pltpu.VMEM((K,), jnp.float32),
            pltpu.VMEM((K, de), jnp.float32),
            pltpu.VMEM((K, de), jnp.float32),
            pltpu.VMEM((TAIL,), jnp.int32),
            pltpu.VMEM((TAIL,), jnp.int32),
            pltpu.VMEM((TAIL, de), jnp.float32),
            pltpu.VMEM_SHARED((N, de), jnp.float32),
            pltpu.SemaphoreType.DMA,
            pltpu.SemaphoreType.DMA,
            pltpu.SemaphoreType.DMA,
            pltpu.SemaphoreType.DMA,
            pltpu.SemaphoreType.DMA,
            pltpu.SemaphoreType.DMA,
        ],
    )
    def agg(h_hbm, scal_hbm, src_hbm, dst_hbm, m_hbm, out_hbm, den_hbm,
            scal_v, m_v, den_v, srcall_v, dst_a, dst_b, w_v, rows_a, rows_b,
            src_t, dst_t, rows_t, acc_sh,
            gsem_a, gsem_b, ssem_a, ssem_b, dsem_a, dsem_b):
        c = lax.axis_index("c")
        s = lax.axis_index("s")
        wid = c * NS + s
        ebase = wid * EPT

        pltpu.sync_copy(scal_hbm, scal_v)
        pltpu.sync_copy(src_hbm.at[pl.ds(ebase, EPT)], srcall_v)
        pltpu.sync_copy(m_hbm, m_v)
        mvec = m_v[...]

        zf = jnp.zeros((L,), jnp.float32)
        oi = jnp.ones((L,), jnp.int32)

        @pl.loop(0, N // L)
        def _(i):
            den_v[pl.ds(i * L, L)] = zf

        @pl.loop(0, K)
        def _(k):
            for ch in range(nch):
                rows_a[k, pl.ds(ch * L, L)] = zf

        nfull = WB // K

        @pl.loop(0, nfull)
        def _(i):
            pltpu.sync_copy(rows_a, acc_sh.at[pl.ds(s * WB + i * K, K)])

        rem = WB - nfull * K
        pltpu.sync_copy(rows_a.at[pl.ds(0, rem)],
                        acc_sh.at[pl.ds(s * WB + nfull * K, rem)])

        @pl.when(s == 0)
        def _():
            pltpu.sync_copy(rows_a.at[pl.ds(0, WB_REM)],
                            acc_sh.at[pl.ds(NS * WB, WB_REM)])

        plsc.subcore_barrier()

        def weights(kb, boff, dstr):
            for j in range(kb // L):
                si = srcall_v[pl.ds(boff + j * L, L)]
                di = dstr[pl.ds(j * L, L)]
                es = plsc.load_gather(scal_v, [si * 2])
                ed = plsc.load_gather(scal_v, [di * 2 + oi])
                e = es + ed
                e = jnp.where(e > 0.0, e, NEG_SLOPE * e)
                w = jnp.exp(e - mvec)
                w_v[pl.ds(j * L, L)] = w
                plsc.addupdate_scatter(den_v, [di], w)

        def scale(kb, rowsr):
            @pl.loop(0, kb // L)
            def _(j):
                wv = w_v[pl.ds(j * L, L)]
                for t in range(L):
                    wk = wv[t]
                    for ch in range(nch):
                        sl = pl.ds(ch * L, L)
                        rowsr[j * L + t, sl] = rowsr[j * L + t, sl] * wk

        def issue_dst(b, dstr, dsem):
            pltpu.async_copy(dst_hbm.at[pl.ds(ebase + b * K, K)], dstr, dsem)

        def issue_gather(b, rowsr, gsem):
            pltpu.async_copy(h_hbm.at[srcall_v.at[pl.ds(b * K, K)]],
                             rowsr, gsem)

        def half(b, dstr, rowsr, gsem, ssem, dsem):
            pltpu.make_async_copy(dst_hbm.at[pl.ds(0, K)], dstr, dsem).wait()
            weights(K, b * K, dstr)
            pltpu.make_async_copy(h_hbm.at[srcall_v.at[pl.ds(0, K)]],
                                  rowsr, gsem).wait()
            scale(K, rowsr)
            pltpu.async_copy(rowsr, acc_sh.at[dstr], ssem, add=True)

        def prefetch(b, dstr, rowsr, gsem, ssem, dsem):
            pltpu.make_async_copy(rowsr, acc_sh.at[dstr], ssem).wait()
            issue_dst(b, dstr, dsem)
            issue_gather(b, rowsr, gsem)

        issue_dst(0, dst_a, dsem_a)
        issue_gather(0, rows_a, gsem_a)
        issue_dst(1, dst_b, dsem_b)
        issue_gather(1, rows_b, gsem_b)

        @pl.loop(0, NB // 2)
        def _(i):
            half(2 * i, dst_a, rows_a, gsem_a, ssem_a, dsem_a)
            half(2 * i + 1, dst_b, rows_b, gsem_b, ssem_b, dsem_b)

            @pl.when(i < NB // 2 - 1)
            def _():
                prefetch(2 * i + 2, dst_a, rows_a, gsem_a, ssem_a, dsem_a)
                prefetch(2 * i + 3, dst_b, rows_b, gsem_b, ssem_b, dsem_b)

        pltpu.make_async_copy(rows_a, acc_sh.at[dst_a], ssem_a).wait()
        pltpu.make_async_copy(rows_b, acc_sh.at[dst_b], ssem_b).wait()

        tbase = ebase + NB * K
        pltpu.sync_copy(src_hbm.at[pl.ds(tbase, TAIL)], src_t)
        pltpu.sync_copy(dst_hbm.at[pl.ds(tbase, TAIL)], dst_t)
        pltpu.sync_copy(h_hbm.at[src_t], rows_t)
        for j in range(TAIL // L):
            si = src_t[pl.ds(j * L, L)]
            di = dst_t[pl.ds(j * L, L)]
            es = plsc.load_gather(scal_v, [si * 2])
            ed = plsc.load_gather(scal_v, [di * 2 + oi])
            e = es + ed
            e = jnp.where(e > 0.0, e, NEG_SLOPE * e)
            w = jnp.exp(e - mvec)
            w_v[pl.ds(j * L, L)] = w
            plsc.addupdate_scatter(den_v, [di], w)

        @pl.loop(0, TAIL // L)
        def _(j):
            wv = w_v[pl.ds(j * L, L)]
            for t in range(L):
                wk = wv[t]
                for ch in range(nch):
                    sl = pl.ds(ch * L, L)
                    rows_t[j * L + t, sl] = rows_t[j * L + t, sl] * wk

        pltpu.sync_copy(rows_t, acc_sh.at[dst_t], add=True)

        pltpu.sync_copy(den_v, den_hbm.at[wid])
        plsc.subcore_barrier()
        pltpu.sync_copy(acc_sh.at[pl.ds(s * WB, WB)],
                        out_hbm.at[c, pl.ds(s * WB, WB)])

        @pl.when(s == 0)
        def _():
            pltpu.sync_copy(acc_sh.at[pl.ds(NS * WB, WB_REM)],
                            out_hbm.at[c, pl.ds(NS * WB, WB_REM)])

    return agg


def _tc_front(x, w1, a1):
    def body(x_ref, w_ref, a_ref, h_ref, scal_ref, m_ref):
        h = jnp.dot(x_ref[...], w_ref[...],
                    preferred_element_type=jnp.float32,
                    precision=lax.Precision.HIGHEST)
        h_ref[...] = h
        sc = jnp.dot(h, a_ref[...], preferred_element_type=jnp.float32,
                     precision=lax.Precision.HIGHEST)
        scal_ref[...] = sc
        m = jnp.maximum(jnp.max(sc[:, 0]) + jnp.max(sc[:, 1]), 0.0)
        m_ref[...] = jnp.full((1, L), m, jnp.float32)

    return pl.pallas_call(
        body,
        out_shape=[
            jax.ShapeDtypeStruct((N, D_HID), jnp.float32),
            jax.ShapeDtypeStruct((N, 2), jnp.float32),
            jax.ShapeDtypeStruct((1, L), jnp.float32),
        ],
    )(x, w1, a1)


def _tc_mid(out1, den1, w2, a2, de2):
    def body(o_ref, d_ref, w_ref, a_ref, h_ref, scal_ref, m_ref):
        t = o_ref[0] + o_ref[1]
        den = jnp.sum(d_ref[...], axis=0)
        h1 = t / jnp.maximum(den, 1e-30)[:, None]
        h1 = jnp.where(h1 > 0.0, h1, jnp.exp(h1) - 1.0)
        h2 = jnp.dot(h1, w_ref[...], preferred_element_type=jnp.float32,
                     precision=lax.Precision.HIGHEST)
        h_ref[...] = jnp.concatenate(
            [h2, jnp.zeros((N, de2 - N_CLASS), jnp.float32)], axis=1)
        sc = jnp.dot(h2, a_ref[...], preferred_element_type=jnp.float32,
                     precision=lax.Precision.HIGHEST)
        scal_ref[...] = sc
        m = jnp.maximum(jnp.max(sc[:, 0]) + jnp.max(sc[:, 1]), 0.0)
        m_ref[...] = jnp.full((1, L), m, jnp.float32)

    return pl.pallas_call(
        body,
        out_shape=[
            jax.ShapeDtypeStruct((N, de2), jnp.float32),
            jax.ShapeDtypeStruct((N, 2), jnp.float32),
            jax.ShapeDtypeStruct((1, L), jnp.float32),
        ],
    )(out1, den1, w2, a2)


def _tc_back(out2, den2):
    def body(o_ref, d_ref, out_ref):
        t = (o_ref[0] + o_ref[1])[:, :N_CLASS]
        den = jnp.sum(d_ref[...], axis=0)
        h2 = t / jnp.maximum(den, 1e-30)[:, None]
        mx = jnp.max(h2, axis=1, keepdims=True)
        z = h2 - mx
        lse = jnp.log(jnp.sum(jnp.exp(z), axis=1, keepdims=True))
        out_ref[...] = z - lse

    return pl.pallas_call(
        body,
        out_shape=jax.ShapeDtypeStruct((N, N_CLASS), jnp.float32),
    )(out2, den2)


def kernel(X, A, W1, a1_src, a1_dst, W2, a2_src, a2_dst):
    src = A[0]
    dst = A[1]
    a1 = jnp.stack([a1_src, a1_dst], axis=1)
    a2 = jnp.stack([a2_src, a2_dst], axis=1)

    h1, scal1, m1 = _tc_front(X, W1, a1)
    out1, den1 = _sc_aggregate(D_HID)(
        h1, scal1.reshape(2 * N), src, dst, m1.reshape(L))

    de2 = 48
    h2, scal2, m2 = _tc_mid(out1, den1, W2, a2, de2)
    out2, den2 = _sc_aggregate(de2)(
        h2, scal2.reshape(2 * N), src, dst, m2.reshape(L))

    return _tc_back(out2, den2)


# speedup vs baseline: 62.4501x; 1.3502x over previous
import dataclasses
import functools

import jax
import jax.numpy as jnp
from jax import lax
from jax.experimental import pallas as pl
from jax.experimental.pallas import tpu as pltpu
from jax.experimental.pallas import tpu_sc as plsc

N = 10000
E = 320000
D_IN = 128
D_HID = 64
N_CLASS = 40
NEG_SLOPE = 0.2

NC = 2
NS = 16
NW = NC * NS
L = 16

EPT = E // NW
K = 128
NB = EPT // K
TAIL = EPT - NB * K
WB = 624
WB_REM = N - NS * WB


def _sc_aggregate(de):
    mesh = plsc.VectorSubcoreMesh(core_axis_name="c", subcore_axis_name="s")
    nch = de // L

    cp = pltpu.CompilerParams()
    for fld, val in (("needs_layout_passes", False),
                     ("use_tc_tiling_on_sc", False)):
        if fld in pltpu.CompilerParams.__dataclass_fields__:
            cp = dataclasses.replace(cp, **{fld: val})

    @functools.partial(
        pl.kernel,
        compiler_params=cp,
        out_type=[
            jax.ShapeDtypeStruct((NC, N, de), jnp.float32),
            jax.ShapeDtypeStruct((NW, N), jnp.float32),
        ],
        mesh=mesh,
        scratch_types=[
            pltpu.VMEM((2 * N,), jnp.float32),
            pltpu.VMEM((16,), jnp.float32),
            pltpu.VMEM((N,), jnp.float32),
            pltpu.VMEM((EPT,), jnp.int32),
            pltpu.VMEM((K,), jnp.int32),
            pltpu.VMEM((K,), jnp.int32),
            pltpu.VMEM((K,), jnp.float32),
            pltpu.VMEM((K, de), jnp.float32),
            pltpu.VMEM((K, de), jnp.float32),
            pltpu.VMEM((TAIL,), jnp.int32),
            pltpu.VMEM((TAIL,), jnp.int32),
            pltpu.VMEM((TAIL, de), jnp.float32),
            pltpu.VMEM_SHARED((N, de), jnp.float32),
            pltpu.SemaphoreType.DMA,
            pltpu.SemaphoreType.DMA,
            pltpu.SemaphoreType.DMA,
            pltpu.SemaphoreType.DMA,
            pltpu.SemaphoreType.DMA,
            pltpu.SemaphoreType.DMA,
        ],
    )
    def agg(h_hbm, scal_hbm, src_hbm, dst_hbm, m_hbm, out_hbm, den_hbm,
            scal_v, m_v, den_v, srcall_v, dst_a, dst_b, w_v, rows_a, rows_b,
            src_t, dst_t, rows_t, acc_sh,
            gsem_a, gsem_b, ssem_a, ssem_b, dsem_a, dsem_b):
        c = lax.axis_index("c")
        s = lax.axis_index("s")
        wid = c * NS + s
        ebase = wid * EPT

        pltpu.sync_copy(scal_hbm, scal_v)
        pltpu.sync_copy(src_hbm.at[pl.ds(ebase, EPT)], srcall_v)
        pltpu.sync_copy(m_hbm, m_v)
        mvec = m_v[...]

        zf = jnp.zeros((L,), jnp.float32)
        oi = jnp.ones((L,), jnp.int32)

        @pl.loop(0, N // L)
        def _(i):
            den_v[pl.ds(i * L, L)] = zf

        @pl.loop(0, K)
        def _(k):
            for ch in range(nch):
                rows_a[k, pl.ds(ch * L, L)] = zf

        nfull = WB // K

        @pl.loop(0, nfull)
        def _(i):
            pltpu.sync_copy(rows_a, acc_sh.at[pl.ds(s * WB + i * K, K)])

        rem = WB - nfull * K
        pltpu.sync_copy(rows_a.at[pl.ds(0, rem)],
                        acc_sh.at[pl.ds(s * WB + nfull * K, rem)])

        @pl.when(s == 0)
        def _():
            pltpu.sync_copy(rows_a.at[pl.ds(0, WB_REM)],
                            acc_sh.at[pl.ds(NS * WB, WB_REM)])

        plsc.subcore_barrier()

        def weights(kb, boff, dstr):
            for j in range(kb // L):
                si = srcall_v[pl.ds(boff + j * L, L)]
                di = dstr[pl.ds(j * L, L)]
                es = plsc.load_gather(scal_v, [si * 2])
                ed = plsc.load_gather(scal_v, [di * 2 + oi])
                e = es + ed
                e = jnp.where(e > 0.0, e, NEG_SLOPE * e)
                w = jnp.exp(e - mvec)
                w_v[pl.ds(j * L, L)] = w
                plsc.addupdate_scatter(den_v, [di], w)

        def scale(kb, rowsr):
            @pl.loop(0, kb // L)
            def _(j):
                wv = w_v[pl.ds(j * L, L)]
                for t in range(L):
                    wk = wv[t]
                    for ch in range(nch):
                        sl = pl.ds(ch * L, L)
                        rowsr[j * L + t, sl] = rowsr[j * L + t, sl] * wk

        def issue_dst(b, dstr, dsem):
            pltpu.async_copy(dst_hbm.at[pl.ds(ebase + b * K, K)], dstr, dsem)

        def issue_gather(b, rowsr, gsem):
            pltpu.async_copy(h_hbm.at[srcall_v.at[pl.ds(b * K, K)]],
                             rowsr, gsem)

        def half(b, dstr, rowsr, gsem, ssem, dsem):
            pltpu.make_async_copy(dst_hbm.at[pl.ds(0, K)], dstr, dsem).wait()
            weights(K, b * K, dstr)
            pltpu.make_async_copy(h_hbm.at[srcall_v.at[pl.ds(0, K)]],
                                  rowsr, gsem).wait()
            scale(K, rowsr)
            pltpu.async_copy(rowsr, acc_sh.at[dstr], ssem, add=True)

        def prefetch(b, dstr, rowsr, gsem, ssem, dsem):
            pltpu.make_async_copy(rowsr, acc_sh.at[dstr], ssem).wait()
            issue_dst(b, dstr, dsem)
            issue_gather(b, rowsr, gsem)

        issue_dst(0, dst_a, dsem_a)
        issue_gather(0, rows_a, gsem_a)
        issue_dst(1, dst_b, dsem_b)
        issue_gather(1, rows_b, gsem_b)

        @pl.loop(0, NB // 2)
        def _(i):
            half(2 * i, dst_a, rows_a, gsem_a, ssem_a, dsem_a)
            half(2 * i + 1, dst_b, rows_b, gsem_b, ssem_b, dsem_b)

            @pl.when(i < NB // 2 - 1)
            def _():
                prefetch(2 * i + 2, dst_a, rows_a, gsem_a, ssem_a, dsem_a)
                prefetch(2 * i + 3, dst_b, rows_b, gsem_b, ssem_b, dsem_b)

        pltpu.make_async_copy(rows_a, acc_sh.at[dst_a], ssem_a).wait()
        pltpu.make_async_copy(rows_b, acc_sh.at[dst_b], ssem_b).wait()

        tbase = ebase + NB * K
        pltpu.sync_copy(src_hbm.at[pl.ds(tbase, TAIL)], src_t)
        pltpu.sync_copy(dst_hbm.at[pl.ds(tbase, TAIL)], dst_t)
        pltpu.sync_copy(h_hbm.at[src_t], rows_t)
        for j in range(TAIL // L):
            si = src_t[pl.ds(j * L, L)]
            di = dst_t[pl.ds(j * L, L)]
            es = plsc.load_gather(scal_v, [si * 2])
            ed = plsc.load_gather(scal_v, [di * 2 + oi])
            e = es + ed
            e = jnp.where(e > 0.0, e, NEG_SLOPE * e)
            w = jnp.exp(e - mvec)
            w_v[pl.ds(j * L, L)] = w
            plsc.addupdate_scatter(den_v, [di], w)

        @pl.loop(0, TAIL // L)
        def _(j):
            wv = w_v[pl.ds(j * L, L)]
            for t in range(L):
                wk = wv[t]
                for ch in range(nch):
                    sl = pl.ds(ch * L, L)
                    rows_t[j * L + t, sl] = rows_t[j * L + t, sl] * wk

        pltpu.sync_copy(rows_t, acc_sh.at[dst_t], add=True)

        pltpu.sync_copy(den_v, den_hbm.at[wid])
        plsc.subcore_barrier()
        pltpu.sync_copy(acc_sh.at[pl.ds(s * WB, WB)],
                        out_hbm.at[c, pl.ds(s * WB, WB)])

        @pl.when(s == 0)
        def _():
            pltpu.sync_copy(acc_sh.at[pl.ds(NS * WB, WB_REM)],
                            out_hbm.at[c, pl.ds(NS * WB, WB_REM)])

    return agg


def _tc_front(x, w1, a1, de1):
    def body(x_ref, w_ref, a_ref, h_ref, scal_ref, m_ref):
        h = jnp.dot(x_ref[...], w_ref[...],
                    preferred_element_type=jnp.float32,
                    precision=lax.Precision.HIGHEST)
        h_ref[...] = jnp.concatenate(
            [h, jnp.zeros((N, de1 - D_HID), jnp.float32)], axis=1)
        sc = jnp.dot(h, a_ref[...], preferred_element_type=jnp.float32,
                     precision=lax.Precision.HIGHEST)
        scal_ref[...] = sc
        m = jnp.maximum(jnp.max(sc[:, 0]) + jnp.max(sc[:, 1]), 0.0)
        m_ref[...] = jnp.full((1, L), m, jnp.float32)

    return pl.pallas_call(
        body,
        out_shape=[
            jax.ShapeDtypeStruct((N, de1), jnp.float32),
            jax.ShapeDtypeStruct((N, 2), jnp.float32),
            jax.ShapeDtypeStruct((1, L), jnp.float32),
        ],
    )(x, w1, a1)


def _tc_mid(out1, den1, w2, a2, de2):
    def body(o_ref, d_ref, w_ref, a_ref, h_ref, scal_ref, m_ref):
        t = (o_ref[0] + o_ref[1])[:, :D_HID]
        den = jnp.sum(d_ref[...], axis=0)
        h1 = t / jnp.maximum(den, 1e-30)[:, None]
        h1 = jnp.where(h1 > 0.0, h1, jnp.exp(h1) - 1.0)
        h2 = jnp.dot(h1, w_ref[...], preferred_element_type=jnp.float32,
                     precision=lax.Precision.HIGHEST)
        h_ref[...] = jnp.concatenate(
            [h2, jnp.zeros((N, de2 - N_CLASS), jnp.float32)], axis=1)
        sc = jnp.dot(h2, a_ref[...], preferred_element_type=jnp.float32,
                     precision=lax.Precision.HIGHEST)
        scal_ref[...] = sc
        m = jnp.maximum(jnp.max(sc[:, 0]) + jnp.max(sc[:, 1]), 0.0)
        m_ref[...] = jnp.full((1, L), m, jnp.float32)

    return pl.pallas_call(
        body,
        out_shape=[
            jax.ShapeDtypeStruct((N, de2), jnp.float32),
            jax.ShapeDtypeStruct((N, 2), jnp.float32),
            jax.ShapeDtypeStruct((1, L), jnp.float32),
        ],
    )(out1, den1, w2, a2)


def _tc_back(out2, den2):
    def body(o_ref, d_ref, out_ref):
        t = (o_ref[0] + o_ref[1])[:, :N_CLASS]
        den = jnp.sum(d_ref[...], axis=0)
        h2 = t / jnp.maximum(den, 1e-30)[:, None]
        mx = jnp.max(h2, axis=1, keepdims=True)
        z = h2 - mx
        lse = jnp.log(jnp.sum(jnp.exp(z), axis=1, keepdims=True))
        out_ref[...] = z - lse

    return pl.pallas_call(
        body,
        out_shape=jax.ShapeDtypeStruct((N, N_CLASS), jnp.float32),
    )(out2, den2)


def kernel(X, A, W1, a1_src, a1_dst, W2, a2_src, a2_dst):
    src = A[0]
    dst = A[1]
    a1 = jnp.stack([a1_src, a1_dst], axis=1)
    a2 = jnp.stack([a2_src, a2_dst], axis=1)

    de1 = 80
    h1, scal1, m1 = _tc_front(X, W1, a1, de1)
    out1, den1 = _sc_aggregate(de1)(
        h1, scal1.reshape(2 * N), src, dst, m1.reshape(L))

    de2 = 48
    h2, scal2, m2 = _tc_mid(out1, den1, W2, a2, de2)
    out2, den2 = _sc_aggregate(de2)(
        h2, scal2.reshape(2 * N), src, dst, m2.reshape(L))

    return _tc_back(out2, den2)
